# Initial kernel scaffold; baseline (speedup 1.0000x reference)
import jax
import jax.numpy as jnp
from jax.experimental import pallas as pl


def kernel(x, table, W, b):
    raise NotImplementedError("write your pallas kernel here")


# speedup vs baseline: 8.4007x; 8.4007x over previous
import functools

import jax
import jax.numpy as jnp
from jax import lax
from jax.experimental import pallas as pl
from jax.experimental.pallas import tpu as pltpu
from jax.experimental.pallas import tpu_sc as plsc

_BATCH = 16384
_HIST = 200
_EMBED = 16
_NC = 2
_NS = 16
_NW = _NC * _NS
_ROWS_PER_W = _BATCH // _NW
_CHUNK_ROWS = 16
_IDX_PER_CHUNK = _CHUNK_ROWS * _HIST
_STREAM = 128
_NSTREAM = _IDX_PER_CHUNK // _STREAM
_NCHUNK = _ROWS_PER_W // _CHUNK_ROWS
_X128 = _BATCH * _HIST // 128


def _sc_body(xf_hbm, table_hbm, p_hbm, out_hbm,
             idx_v, emb_v, p_v, out_v, sem_i, sem_g):
    wid = lax.axis_index("s") * _NC + lax.axis_index("c")
    pltpu.sync_copy(p_hbm, p_v)
    base_idx = wid * (_ROWS_PER_W * _HIST)
    rows16 = lax.iota(jnp.int32, 16)
    wv = p_v[pl.ds(0, 16)]
    bias = p_v[pl.ds(16, 16)][0]

    def chunk_body(c, carry):
        ioff = base_idx + c * _IDX_PER_CHUNK
        pltpu.async_copy(
            xf_hbm.at[pl.ds(ioff, _IDX_PER_CHUNK)], idx_v, sem_i).wait()
        for j in range(_NSTREAM):
            pltpu.async_copy(
                table_hbm.at[idx_v.at[pl.ds(j * _STREAM, _STREAM)]],
                emb_v.at[pl.ds(j * _STREAM, _STREAM), :],
                sem_g)
        pltpu.make_async_copy(
            table_hbm.at[pl.ds(0, _IDX_PER_CHUNK), :], emb_v, sem_g).wait()

        def lbody(l, accs):
            return tuple(accs[r] + emb_v[r * _HIST + l, :]
                         for r in range(_CHUNK_ROWS))
        accs = lax.fori_loop(
            0, _HIST, lbody,
            tuple(jnp.zeros((16,), jnp.float32) for _ in range(_CHUNK_ROWS)))
        s = jnp.zeros((16,), jnp.float32)
        for r in range(_CHUNK_ROWS):
            prod = accs[r] * wv
            sr = prod[0]
            for d in range(1, _EMBED):
                sr = sr + prod[d]
            s = jnp.where(rows16 == r, sr, s)
        y = 1.0 / (1.0 + jnp.exp(-(s + bias)))
        y = (y * 10000.0 + 0.5).astype(jnp.int32).astype(jnp.float32) * 1e-4
        out_v[pl.ds(c * _CHUNK_ROWS, _CHUNK_ROWS)] = y
        return carry

    lax.fori_loop(0, _NCHUNK, chunk_body, 0)
    pltpu.sync_copy(out_v, out_hbm.at[pl.ds(wid * _ROWS_PER_W, _ROWS_PER_W)])


@functools.partial(jax.jit, static_argnums=())
def _launch(xf, table, p):
    mesh = plsc.VectorSubcoreMesh(core_axis_name="c", subcore_axis_name="s")
    f = functools.partial(
        pl.kernel,
        out_type=jax.ShapeDtypeStruct((_BATCH,), jnp.float32),
        mesh=mesh,
        compiler_params=pltpu.CompilerParams(use_tc_tiling_on_sc=False),
        scratch_types=[
            pltpu.VMEM((_IDX_PER_CHUNK,), jnp.int32),
            pltpu.VMEM((_IDX_PER_CHUNK, _EMBED), jnp.float32),
            pltpu.VMEM((32,), jnp.float32),
            pltpu.VMEM((_ROWS_PER_W,), jnp.float32),
            pltpu.SemaphoreType.DMA,
            pltpu.SemaphoreType.DMA,
        ],
    )(_sc_body)
    return f(xf, table, p)


def kernel(x, table, W, b):
    xf = x.astype(jnp.int32).reshape(_BATCH * _HIST)
    p = jnp.concatenate([
        W.reshape(_EMBED).astype(jnp.float32) / float(_HIST),
        b.reshape(1).astype(jnp.float32),
        jnp.zeros((15,), jnp.float32),
    ])
    out = _launch(xf, table, p)
    return out.reshape(_BATCH, 1)
